# Initial kernel scaffold; baseline (speedup 1.0000x reference)
import jax
import jax.numpy as jnp
from jax.experimental import pallas as pl


def kernel(X_in):
    raise NotImplementedError("write your pallas kernel here")


# speedup vs baseline: 9.6048x; 9.6048x over previous
import jax
import jax.numpy as jnp
from jax import lax
from jax.experimental import pallas as pl
from jax.experimental.pallas import tpu as pltpu
from jax.experimental.pallas import tpu_sc as plsc

_B, _L, _D, _N = 8, 2048, 16, 32
_C = _D * _N
_NC, _NS = 2, 16
_NW = _NC * _NS
_CPW = _C // _NW


def _scan_body(x_hbm, out_hbm, ibuf, obuf):
    wid = lax.axis_index("s") * _NC + lax.axis_index("c")
    c0 = wid * _CPW
    for b in range(_B):
        pltpu.sync_copy(x_hbm.at[b, :, pl.ds(c0, _CPW)], ibuf)

        def step(i, acc):
            a = acc + ibuf[i]
            obuf[i] = a
            return a

        lax.fori_loop(0, _L, step, jnp.zeros((_CPW,), jnp.float32))
        pltpu.sync_copy(obuf, out_hbm.at[b, :, pl.ds(c0, _CPW)])


@jax.jit
def kernel(X_in):
    x = X_in.reshape(_B, _L, _C)
    run = pl.kernel(
        _scan_body,
        out_type=jax.ShapeDtypeStruct((_B, _L, _C), jnp.float32),
        scratch_types=[
            pltpu.VMEM((_L, _CPW), jnp.float32),
            pltpu.VMEM((_L, _CPW), jnp.float32),
        ],
        mesh=plsc.VectorSubcoreMesh(core_axis_name="c", subcore_axis_name="s"),
        compiler_params=pltpu.CompilerParams(use_tc_tiling_on_sc=False),
    )
    return run(x).reshape(_B, _L, _D, _N)


# speedup vs baseline: 13.4626x; 1.4017x over previous
import jax
import jax.numpy as jnp
from jax import lax
from jax.experimental import pallas as pl
from jax.experimental.pallas import tpu as pltpu
from jax.experimental.pallas import tpu_sc as plsc

_B, _L, _D, _N = 8, 2048, 16, 32
_C = _D * _N
_NC, _NS = 2, 16
_NW = _NC * _NS
_CPW = _C // _NW
_CH = 128
_NCH = _L // _CH


def _scan_body(x_hbm, out_hbm, ib0, ib1, ob0, ob1, is0, is1, os0, os1):
    wid = lax.axis_index("s") * _NC + lax.axis_index("c")
    c0 = wid * _CPW
    ibufs, obufs = (ib0, ib1), (ob0, ob1)
    isems, osems = (is0, is1), (os0, os1)

    def in_cp(g):
        return pltpu.async_copy(
            x_hbm.at[:, pl.ds(g * _CH, _CH), pl.ds(c0, _CPW)],
            ibufs[g % 2], isems[g % 2])

    def out_cp(g):
        return pltpu.async_copy(
            obufs[g % 2],
            out_hbm.at[:, pl.ds(g * _CH, _CH), pl.ds(c0, _CPW)],
            osems[g % 2])

    in_h = [in_cp(0), None]
    out_h = [None, None]
    accs = tuple(jnp.zeros((_CPW,), jnp.float32) for _ in range(_B))
    for g in range(_NCH):
        s = g % 2
        if g + 1 < _NCH:
            in_h[(g + 1) % 2] = in_cp(g + 1)
        in_h[s].wait()
        if out_h[s] is not None:
            out_h[s].wait()
        ib, ob = ibufs[s], obufs[s]

        def step(i, acc_t):
            new = []
            for b in range(_B):
                a = acc_t[b] + ib[b, i]
                ob[b, i] = a
                new.append(a)
            return tuple(new)

        accs = lax.fori_loop(0, _CH, step, accs)
        out_h[s] = out_cp(g)
    out_h[0].wait()
    out_h[1].wait()


@jax.jit
def kernel(X_in):
    x = X_in.reshape(_B, _L, _C)
    run = pl.kernel(
        _scan_body,
        out_type=jax.ShapeDtypeStruct((_B, _L, _C), jnp.float32),
        scratch_types=[
            pltpu.VMEM((_B, _CH, _CPW), jnp.float32),
            pltpu.VMEM((_B, _CH, _CPW), jnp.float32),
            pltpu.VMEM((_B, _CH, _CPW), jnp.float32),
            pltpu.VMEM((_B, _CH, _CPW), jnp.float32),
            pltpu.SemaphoreType.DMA,
            pltpu.SemaphoreType.DMA,
            pltpu.SemaphoreType.DMA,
            pltpu.SemaphoreType.DMA,
        ],
        mesh=plsc.VectorSubcoreMesh(core_axis_name="c", subcore_axis_name="s"),
        compiler_params=pltpu.CompilerParams(use_tc_tiling_on_sc=False),
    )
    return run(x).reshape(_B, _L, _D, _N)


# speedup vs baseline: 29.3461x; 2.1798x over previous
import jax
import jax.numpy as jnp
from jax import lax
from jax.experimental import pallas as pl
from jax.experimental.pallas import tpu as pltpu
from jax.experimental.pallas import tpu_sc as plsc

_B, _L, _D, _N = 8, 2048, 16, 32
_NC, _NS = 2, 16
_NW = _NC * _NS
_NO = _N // 8
_UNITS = _B * _D * _NO
_UPW = _UNITS // _NW
_NVR = _L // 16


def _unit_slice(ref, u):
    b = u // (_D * _NO)
    d = (u % (_D * _NO)) // _NO
    no = u % _NO
    return ref.at[b, d, pl.ds(no * 8, 8), :]


def _scan_body(x_hbm, out_hbm, ib0, ib1, ob0, ob1, is0, is1, os0, os1):
    wid = lax.axis_index("s") * _NC + lax.axis_index("c")
    u0 = wid * _UPW
    ibufs, obufs = (ib0, ib1), (ob0, ob1)
    isems, osems = (is0, is1), (os0, os1)

    def in_cp(k):
        return pltpu.async_copy(
            _unit_slice(x_hbm, u0 + k), ibufs[k % 2], isems[k % 2])

    def out_cp(k):
        return pltpu.async_copy(
            obufs[k % 2], _unit_slice(out_hbm, u0 + k), osems[k % 2])

    in_h = [in_cp(0), None]
    out_h = [None, None]
    for k in range(_UPW):
        s = k % 2
        if k + 1 < _UPW:
            in_h[(k + 1) % 2] = in_cp(k + 1)
        in_h[s].wait()
        if out_h[s] is not None:
            out_h[s].wait()
        ib, ob = ibufs[s], obufs[s]

        def step(j, carry):
            new = []
            for r in range(8):
                v = ib[r, pl.ds(j * 16, 16)]
                out = plsc.cumsum(v) + carry[r]
                ob[r, pl.ds(j * 16, 16)] = out
                new.append(jnp.broadcast_to(out[15], (16,)))
            return tuple(new)

        lax.fori_loop(0, _NVR, step,
                      tuple(jnp.zeros((16,), jnp.float32) for _ in range(8)))
        out_h[s] = out_cp(k)
    out_h[0].wait()
    out_h[1].wait()


@jax.jit
def kernel(X_in):
    xt = jnp.transpose(X_in, (0, 2, 3, 1))
    run = pl.kernel(
        _scan_body,
        out_type=jax.ShapeDtypeStruct((_B, _D, _N, _L), jnp.float32),
        scratch_types=[
            pltpu.VMEM((8, _L), jnp.float32),
            pltpu.VMEM((8, _L), jnp.float32),
            pltpu.VMEM((8, _L), jnp.float32),
            pltpu.VMEM((8, _L), jnp.float32),
            pltpu.SemaphoreType.DMA,
            pltpu.SemaphoreType.DMA,
            pltpu.SemaphoreType.DMA,
            pltpu.SemaphoreType.DMA,
        ],
        mesh=plsc.VectorSubcoreMesh(core_axis_name="c", subcore_axis_name="s"),
        compiler_params=pltpu.CompilerParams(needs_layout_passes=False),
    )
    return jnp.transpose(run(xt), (0, 3, 1, 2))


# speedup vs baseline: 30.4627x; 1.0380x over previous
import jax
import jax.numpy as jnp
from jax import lax
from jax.experimental import pallas as pl
from jax.experimental.pallas import tpu as pltpu
from jax.experimental.pallas import tpu_sc as plsc

_B, _L, _D, _N = 8, 2048, 16, 32
_NC, _NS = 2, 16
_NW = _NC * _NS
_NH = _N // 16
_UNITS = _B * _D * _NH
_UPW = _UNITS // _NW
_NVR = _L // 16


def _unit_slice(ref, u):
    b = u // (_D * _NH)
    d = (u % (_D * _NH)) // _NH
    nh = u % _NH
    return ref.at[b, d, pl.ds(nh * 16, 16), :]


def _scan_body(x_hbm, out_hbm, buf0, buf1, is0, is1, os0, os1):
    wid = lax.axis_index("s") * _NC + lax.axis_index("c")
    u0 = wid * _UPW
    bufs = (buf0, buf1)
    isems, osems = (is0, is1), (os0, os1)

    def in_cp(k):
        return pltpu.async_copy(
            _unit_slice(x_hbm, u0 + k), bufs[k % 2], isems[k % 2])

    def out_cp(k):
        return pltpu.async_copy(
            bufs[k % 2], _unit_slice(out_hbm, u0 + k), osems[k % 2])

    in_h = [in_cp(0), None]
    out_h = [None, None]
    for k in range(_UPW):
        s = k % 2
        if k + 1 < _UPW:
            if out_h[(k + 1) % 2] is not None:
                out_h[(k + 1) % 2].wait()
                out_h[(k + 1) % 2] = None
            in_h[(k + 1) % 2] = in_cp(k + 1)
        in_h[s].wait()
        ib = bufs[s]

        def step(j, carry):
            new = []
            for r in range(16):
                v = ib[r, pl.ds(j * 16, 16)]
                out = plsc.cumsum(v) + carry[r]
                ib[r, pl.ds(j * 16, 16)] = out
                new.append(jnp.broadcast_to(out[15], (16,)))
            return tuple(new)

        lax.fori_loop(0, _NVR, step,
                      tuple(jnp.zeros((16,), jnp.float32) for _ in range(16)))
        out_h[s] = out_cp(k)
    out_h[0].wait()
    out_h[1].wait()


@jax.jit
def kernel(X_in):
    xt = jnp.transpose(X_in, (0, 2, 3, 1))
    run = pl.kernel(
        _scan_body,
        out_type=jax.ShapeDtypeStruct((_B, _D, _N, _L), jnp.float32),
        scratch_types=[
            pltpu.VMEM((16, _L), jnp.float32),
            pltpu.VMEM((16, _L), jnp.float32),
            pltpu.SemaphoreType.DMA,
            pltpu.SemaphoreType.DMA,
            pltpu.SemaphoreType.DMA,
            pltpu.SemaphoreType.DMA,
        ],
        mesh=plsc.VectorSubcoreMesh(core_axis_name="c", subcore_axis_name="s"),
        compiler_params=pltpu.CompilerParams(needs_layout_passes=False),
    )
    return jnp.transpose(run(xt), (0, 3, 1, 2))
